# Initial kernel scaffold; baseline (speedup 1.0000x reference)
import jax
import jax.numpy as jnp
from jax.experimental import pallas as pl


def kernel(x, block, W1, b1, W2, b2):
    raise NotImplementedError("write your pallas kernel here")


# speedup vs baseline: 15.5034x; 15.5034x over previous
import functools

import jax
import jax.numpy as jnp
from jax import lax
from jax.experimental import pallas as pl
from jax.experimental.pallas import tpu as pltpu
from jax.experimental.pallas import tpu_sc as plsc

N = 10000
E = 320000
F_IN = 128
F_HID = 256
F_OUT = 64

NC = 2
NS = 16
NW = NC * NS
EPW = E // NW
K = 80
NCHUNK = EPW // K
R0 = 624
R1 = N - 15 * R0


def _rowwise_copy(s, mk_src, mk_dst):
    @pl.when(s < 15)
    def _():
        pltpu.sync_copy(mk_src(s * R0, R0), mk_dst(s * R0, R0))

    @pl.when(s == 15)
    def _():
        pltpu.sync_copy(mk_src(15 * R0, R1), mk_dst(15 * R0, R1))

@functools.cache
def _get_sc_degree():
    mesh = plsc.VectorSubcoreMesh(core_axis_name="c", subcore_axis_name="s")

    @functools.partial(
        pl.kernel,
        out_type=jax.ShapeDtypeStruct((2 * N, 8), jnp.float32),
        mesh=mesh,
        scratch_types=[
            pltpu.VMEM((K,), jnp.int32),
            pltpu.VMEM((K, 8), jnp.float32),
            pltpu.VMEM_SHARED((N, 8), jnp.float32),
            pltpu.SemaphoreType.DMA,
        ],
        compiler_params=pltpu.CompilerParams(use_tc_tiling_on_sc=False),
    )
    def _sc_degree(dst_hbm, ones_hbm, out_hbm, idx_v, ones_v, acc_sh, sem):
        c = lax.axis_index("c")
        s = lax.axis_index("s")
        _rowwise_copy(s, lambda o, n: ones_hbm.at[pl.ds(o, n)],
                      lambda o, n: acc_sh.at[pl.ds(o, n)])
        pltpu.sync_copy(ones_hbm.at[pl.ds(0, K)], ones_v)
        plsc.subcore_barrier()

        wid = c * NS + s

        def chunk(i, carry):
            base = wid * EPW + i * K
            pltpu.sync_copy(dst_hbm.at[pl.ds(base, K)], idx_v)
            pltpu.sync_copy(ones_v, acc_sh.at[idx_v], add=True)
            return carry

        lax.fori_loop(0, NCHUNK, chunk, 0)
        plsc.subcore_barrier()
        _rowwise_copy(s, lambda o, n: acc_sh.at[pl.ds(o, n)],
                      lambda o, n: out_hbm.at[pl.ds(c * N + o, n)])

    return _sc_degree


@functools.cache
def _make_sc_prop(F):
    mesh = plsc.VectorSubcoreMesh(core_axis_name="c", subcore_axis_name="s")

    @functools.partial(
        pl.kernel,
        out_type=jax.ShapeDtypeStruct((2 * N, F), jnp.float32),
        mesh=mesh,
        scratch_types=[
            pltpu.VMEM((K,), jnp.int32),
            pltpu.VMEM((K,), jnp.int32),
            pltpu.VMEM((K, F), jnp.float32),
            pltpu.VMEM_SHARED((N, F), jnp.float32),
            pltpu.SemaphoreType.DMA,
        ],
        compiler_params=pltpu.CompilerParams(use_tc_tiling_on_sc=False),
    )
    def _sc_prop(feat_hbm, src_hbm, dst_hbm, out_hbm, sidx, didx, rows, acc_sh, sem):
        c = lax.axis_index("c")
        s = lax.axis_index("s")
        _rowwise_copy(s, lambda o, n: feat_hbm.at[pl.ds(o, n)],
                      lambda o, n: acc_sh.at[pl.ds(o, n)])
        plsc.subcore_barrier()

        wid = c * NS + s

        def chunk(i, carry):
            base = wid * EPW + i * K
            pltpu.sync_copy(src_hbm.at[pl.ds(base, K)], sidx)
            pltpu.sync_copy(dst_hbm.at[pl.ds(base, K)], didx)
            pltpu.async_copy(feat_hbm.at[sidx], rows, sem).wait()
            pltpu.sync_copy(rows, acc_sh.at[didx], add=True)
            return carry

        lax.fori_loop(0, NCHUNK, chunk, 0)
        plsc.subcore_barrier()
        _rowwise_copy(s, lambda o, n: acc_sh.at[pl.ds(o, n)],
                      lambda o, n: out_hbm.at[pl.ds(c * N + o, n)])

    return _sc_prop


_B = 2000


def _tc_prep_body(deg_ref, x_ref, d_ref, x1_ref):
    deg = deg_ref[0] + deg_ref[1] - 1.0
    d = lax.rsqrt(deg)
    d_ref[...] = d
    x1_ref[...] = x_ref[...] * d[:, :1]


def _tc_prep(deg_par, x):
    return pl.pallas_call(
        _tc_prep_body,
        grid=(N // _B,),
        in_specs=[
            pl.BlockSpec((2, _B, 8), lambda i: (0, i, 0)),
            pl.BlockSpec((_B, F_IN), lambda i: (i, 0)),
        ],
        out_specs=[
            pl.BlockSpec((_B, 8), lambda i: (i, 0)),
            pl.BlockSpec((_B, F_IN), lambda i: (i, 0)),
        ],
        out_shape=[
            jax.ShapeDtypeStruct((N, 8), jnp.float32),
            jax.ShapeDtypeStruct((N, F_IN), jnp.float32),
        ],
    )(deg_par, x)


def _tc_mid_body(p1_ref, x1_ref, d_ref, W1_ref, b1_ref, W2_ref, t2_ref):
    d = d_ref[:, :1]
    p1 = (p1_ref[0] + p1_ref[1] - x1_ref[...]) * d
    h1 = jnp.dot(p1, W1_ref[...], preferred_element_type=jnp.float32)
    h1 = jnp.maximum(h1 + b1_ref[...], 0.0)
    t2 = jnp.dot(h1, W2_ref[...], preferred_element_type=jnp.float32)
    t2_ref[...] = t2 * d


def _tc_mid(p1_par, x1, d8, W1, b1, W2):
    return pl.pallas_call(
        _tc_mid_body,
        grid=(N // _B,),
        in_specs=[
            pl.BlockSpec((2, _B, F_IN), lambda i: (0, i, 0)),
            pl.BlockSpec((_B, F_IN), lambda i: (i, 0)),
            pl.BlockSpec((_B, 8), lambda i: (i, 0)),
            pl.BlockSpec((F_IN, F_HID), lambda i: (0, 0)),
            pl.BlockSpec((1, F_HID), lambda i: (0, 0)),
            pl.BlockSpec((F_HID, F_OUT), lambda i: (0, 0)),
        ],
        out_specs=pl.BlockSpec((_B, F_OUT), lambda i: (i, 0)),
        out_shape=jax.ShapeDtypeStruct((N, F_OUT), jnp.float32),
    )(p1_par, x1, d8, W1, b1.reshape(1, F_HID), W2)


def _tc_final_body(p2_ref, t2_ref, d_ref, b2_ref, out_ref):
    d = d_ref[:, :1]
    p2 = (p2_ref[0] + p2_ref[1] - t2_ref[...]) * d + b2_ref[...]
    m = jnp.max(p2, axis=1, keepdims=True)
    lse = jnp.log(jnp.sum(jnp.exp(p2 - m), axis=1, keepdims=True))
    out_ref[...] = p2 - m - lse


def _tc_final(p2_par, t2, d8, b2):
    return pl.pallas_call(
        _tc_final_body,
        grid=(N // _B,),
        in_specs=[
            pl.BlockSpec((2, _B, F_OUT), lambda i: (0, i, 0)),
            pl.BlockSpec((_B, F_OUT), lambda i: (i, 0)),
            pl.BlockSpec((_B, 8), lambda i: (i, 0)),
            pl.BlockSpec((1, F_OUT), lambda i: (0, 0)),
        ],
        out_specs=pl.BlockSpec((_B, F_OUT), lambda i: (i, 0)),
        out_shape=jax.ShapeDtypeStruct((N, F_OUT), jnp.float32),
    )(p2_par, t2, d8, b2.reshape(1, F_OUT))


def kernel(x, block, W1, b1, W2, b2):
    src = block[0]
    dst = block[1]
    ones8 = jnp.ones((N, 8), dtype=jnp.float32)

    deg_par = _get_sc_degree()(dst, ones8).reshape(2, N, 8)
    d8, x1 = _tc_prep(deg_par, x)
    p1_par = _make_sc_prop(F_IN)(x1, src, dst).reshape(2, N, F_IN)
    t2 = _tc_mid(p1_par, x1, d8, W1, b1, W2)
    p2_par = _make_sc_prop(F_OUT)(t2, src, dst).reshape(2, N, F_OUT)
    return _tc_final(p2_par, t2, d8, b2)


# speedup vs baseline: 39.4081x; 2.5419x over previous
import functools

import jax
import jax.numpy as jnp
from jax import lax
from jax.experimental import pallas as pl
from jax.experimental.pallas import tpu as pltpu
from jax.experimental.pallas import tpu_sc as plsc

N = 10000
E = 320000
F_IN = 128
F_HID = 256
F_OUT = 64

NC = 2
NS = 16
NW = NC * NS
EPW = E // NW
K = 40
NCHUNK = EPW // K
GROUP = 5
NG = NCHUNK // GROUP
R0 = 624
R1 = N - 15 * R0


def _rowwise_copy(s, mk_src, mk_dst):
    @pl.when(s < 15)
    def _():
        pltpu.sync_copy(mk_src(s * R0, R0), mk_dst(s * R0, R0))

    @pl.when(s == 15)
    def _():
        pltpu.sync_copy(mk_src(15 * R0, R1), mk_dst(15 * R0, R1))

@functools.cache
def _get_sc_degree():
    mesh = plsc.VectorSubcoreMesh(core_axis_name="c", subcore_axis_name="s")

    @functools.partial(
        pl.kernel,
        out_type=jax.ShapeDtypeStruct((2 * N, 8), jnp.float32),
        mesh=mesh,
        scratch_types=[
            pltpu.VMEM((NCHUNK, K), jnp.int32),
            pltpu.VMEM((K, 8), jnp.float32),
            pltpu.VMEM_SHARED((N, 8), jnp.float32),
            pltpu.SemaphoreType.DMA,
        ],
        compiler_params=pltpu.CompilerParams(use_tc_tiling_on_sc=False),
    )
    def _sc_degree(dst3_hbm, ones_hbm, out_hbm, didx, ones_v, acc_sh, ssem):
        c = lax.axis_index("c")
        s = lax.axis_index("s")
        wid = c * NS + s
        pltpu.sync_copy(dst3_hbm.at[wid], didx)
        _rowwise_copy(s, lambda o, n: ones_hbm.at[pl.ds(o, n)],
                      lambda o, n: acc_sh.at[pl.ds(o, n)])
        pltpu.sync_copy(ones_hbm.at[pl.ds(0, K)], ones_v)
        plsc.subcore_barrier()

        def scatters(g, issue):
            for b in range(GROUP):
                d = pltpu.make_async_copy(
                    ones_v, acc_sh.at[didx.at[g * GROUP + b]], ssem)
                d.start(add=True) if issue else d.wait()

        def body(g, carry):
            scatters(g, True)
            scatters(g, False)
            return carry

        lax.fori_loop(0, NG, body, 0)
        plsc.subcore_barrier()
        _rowwise_copy(s, lambda o, n: acc_sh.at[pl.ds(o, n)],
                      lambda o, n: out_hbm.at[pl.ds(c * N + o, n)])

    return _sc_degree


@functools.cache
def _make_sc_prop(F, group):
    ng = NCHUNK // group
    mesh = plsc.VectorSubcoreMesh(core_axis_name="c", subcore_axis_name="s")

    @functools.partial(
        pl.kernel,
        out_type=jax.ShapeDtypeStruct((2 * N, F), jnp.float32),
        mesh=mesh,
        scratch_types=[
            pltpu.VMEM((NCHUNK, K), jnp.int32),
            pltpu.VMEM((NCHUNK, K), jnp.int32),
            pltpu.VMEM((2, group, K, F), jnp.float32),
            pltpu.VMEM_SHARED((N, F), jnp.float32),
            pltpu.SemaphoreType.DMA,
            pltpu.SemaphoreType.DMA,
            pltpu.SemaphoreType.DMA,
        ],
        compiler_params=pltpu.CompilerParams(use_tc_tiling_on_sc=False),
    )
    def _sc_prop(feat_hbm, src3_hbm, dst3_hbm, out_hbm,
                 sidx, didx, rows, acc_sh, gsem0, gsem1, ssem):
        c = lax.axis_index("c")
        s = lax.axis_index("s")
        wid = c * NS + s
        pltpu.sync_copy(src3_hbm.at[wid], sidx)
        pltpu.sync_copy(dst3_hbm.at[wid], didx)
        _rowwise_copy(s, lambda o, n: feat_hbm.at[pl.ds(o, n)],
                      lambda o, n: acc_sh.at[pl.ds(o, n)])
        plsc.subcore_barrier()

        gsems = (gsem0, gsem1)

        def gathers(g, h, issue):
            for b in range(group):
                d = pltpu.make_async_copy(
                    feat_hbm.at[sidx.at[g * group + b]], rows.at[h, b], gsems[h])
                d.start() if issue else d.wait()

        def scatters(g, h, issue):
            for b in range(group):
                d = pltpu.make_async_copy(
                    rows.at[h, b], acc_sh.at[didx.at[g * group + b]], ssem)
                d.start(add=True) if issue else d.wait()

        def step(g, h):
            @pl.when(g + 1 < ng)
            def _():
                gathers(g + 1, 1 - h, True)

            gathers(g, h, False)
            scatters(g, h, True)
            scatters(g, h, False)

        gathers(0, 0, True)

        def body(i, carry):
            for h in (0, 1):
                step(i * 2 + h, h)
            return carry

        lax.fori_loop(0, ng // 2, body, 0)
        if ng % 2 == 1:
            step(ng - 1, (ng - 1) % 2)
        plsc.subcore_barrier()
        _rowwise_copy(s, lambda o, n: acc_sh.at[pl.ds(o, n)],
                      lambda o, n: out_hbm.at[pl.ds(c * N + o, n)])

    return _sc_prop


_B = 2000


def _tc_prep_body(deg_ref, x_ref, d_ref, x1_ref):
    deg = deg_ref[0] + deg_ref[1] - 1.0
    d = lax.rsqrt(deg)
    d_ref[...] = d
    x1_ref[...] = x_ref[...] * d[:, :1]


def _tc_prep(deg_par, x):
    return pl.pallas_call(
        _tc_prep_body,
        grid=(N // _B,),
        in_specs=[
            pl.BlockSpec((2, _B, 8), lambda i: (0, i, 0)),
            pl.BlockSpec((_B, F_IN), lambda i: (i, 0)),
        ],
        out_specs=[
            pl.BlockSpec((_B, 8), lambda i: (i, 0)),
            pl.BlockSpec((_B, F_IN), lambda i: (i, 0)),
        ],
        out_shape=[
            jax.ShapeDtypeStruct((N, 8), jnp.float32),
            jax.ShapeDtypeStruct((N, F_IN), jnp.float32),
        ],
    )(deg_par, x)


def _tc_mid_body(p1_ref, x1_ref, d_ref, W1_ref, b1_ref, W2_ref, t2_ref):
    d = d_ref[:, :1]
    p1 = (p1_ref[0] + p1_ref[1] - x1_ref[...]) * d
    h1 = jnp.dot(p1, W1_ref[...], preferred_element_type=jnp.float32)
    h1 = jnp.maximum(h1 + b1_ref[...], 0.0)
    t2 = jnp.dot(h1, W2_ref[...], preferred_element_type=jnp.float32)
    t2_ref[...] = t2 * d


def _tc_mid(p1_par, x1, d8, W1, b1, W2):
    return pl.pallas_call(
        _tc_mid_body,
        grid=(N // _B,),
        in_specs=[
            pl.BlockSpec((2, _B, F_IN), lambda i: (0, i, 0)),
            pl.BlockSpec((_B, F_IN), lambda i: (i, 0)),
            pl.BlockSpec((_B, 8), lambda i: (i, 0)),
            pl.BlockSpec((F_IN, F_HID), lambda i: (0, 0)),
            pl.BlockSpec((1, F_HID), lambda i: (0, 0)),
            pl.BlockSpec((F_HID, F_OUT), lambda i: (0, 0)),
        ],
        out_specs=pl.BlockSpec((_B, F_OUT), lambda i: (i, 0)),
        out_shape=jax.ShapeDtypeStruct((N, F_OUT), jnp.float32),
    )(p1_par, x1, d8, W1, b1.reshape(1, F_HID), W2)


def _tc_final_body(p2_ref, t2_ref, d_ref, b2_ref, out_ref):
    d = d_ref[:, :1]
    p2 = (p2_ref[0] + p2_ref[1] - t2_ref[...]) * d + b2_ref[...]
    m = jnp.max(p2, axis=1, keepdims=True)
    lse = jnp.log(jnp.sum(jnp.exp(p2 - m), axis=1, keepdims=True))
    out_ref[...] = p2 - m - lse


def _tc_final(p2_par, t2, d8, b2):
    return pl.pallas_call(
        _tc_final_body,
        grid=(N // _B,),
        in_specs=[
            pl.BlockSpec((2, _B, F_OUT), lambda i: (0, i, 0)),
            pl.BlockSpec((_B, F_OUT), lambda i: (i, 0)),
            pl.BlockSpec((_B, 8), lambda i: (i, 0)),
            pl.BlockSpec((1, F_OUT), lambda i: (0, 0)),
        ],
        out_specs=pl.BlockSpec((_B, F_OUT), lambda i: (i, 0)),
        out_shape=jax.ShapeDtypeStruct((N, F_OUT), jnp.float32),
    )(p2_par, t2, d8, b2.reshape(1, F_OUT))


def kernel(x, block, W1, b1, W2, b2):
    src3 = block[0].reshape(NW, NCHUNK, K)
    dst3 = block[1].reshape(NW, NCHUNK, K)
    ones8 = jnp.ones((N, 8), dtype=jnp.float32)

    deg_par = _get_sc_degree()(dst3, ones8).reshape(2, N, 8)
    d8, x1 = _tc_prep(deg_par, x)
    p1_par = _make_sc_prop(F_IN, 2)(x1, src3, dst3).reshape(2, N, F_IN)
    t2 = _tc_mid(p1_par, x1, d8, W1, b1, W2)
    p2_par = _make_sc_prop(F_OUT, 5)(t2, src3, dst3).reshape(2, N, F_OUT)
    return _tc_final(p2_par, t2, d8, b2)


# speedup vs baseline: 44.6338x; 1.1326x over previous
import functools

import jax
import jax.numpy as jnp
from jax import lax
from jax.experimental import pallas as pl
from jax.experimental.pallas import tpu as pltpu
from jax.experimental.pallas import tpu_sc as plsc

N = 10000
E = 320000
F_IN = 128
F_HID = 256
F_OUT = 64

NC = 2
NS = 16
NW = NC * NS
EPW = E // NW
K = 40
NCHUNK = EPW // K
GROUP = 5
NG = NCHUNK // GROUP
R0 = 624
R1 = N - 15 * R0


def _rowwise_copy(s, mk_src, mk_dst):
    @pl.when(s < 15)
    def _():
        pltpu.sync_copy(mk_src(s * R0, R0), mk_dst(s * R0, R0))

    @pl.when(s == 15)
    def _():
        pltpu.sync_copy(mk_src(15 * R0, R1), mk_dst(15 * R0, R1))

@functools.cache
def _get_sc_degree():
    mesh = plsc.VectorSubcoreMesh(core_axis_name="c", subcore_axis_name="s")

    @functools.partial(
        pl.kernel,
        out_type=jax.ShapeDtypeStruct((2 * N, 8), jnp.float32),
        mesh=mesh,
        scratch_types=[
            pltpu.VMEM((NCHUNK, K), jnp.int32),
            pltpu.VMEM((K, 8), jnp.float32),
            pltpu.VMEM_SHARED((N, 8), jnp.float32),
            pltpu.SemaphoreType.DMA,
        ],
        compiler_params=pltpu.CompilerParams(use_tc_tiling_on_sc=False),
    )
    def _sc_degree(dst3_hbm, ones_hbm, out_hbm, didx, ones_v, acc_sh, ssem):
        c = lax.axis_index("c")
        s = lax.axis_index("s")
        wid = c * NS + s
        pltpu.sync_copy(dst3_hbm.at[wid], didx)
        _rowwise_copy(s, lambda o, n: ones_hbm.at[pl.ds(o, n)],
                      lambda o, n: acc_sh.at[pl.ds(o, n)])
        pltpu.sync_copy(ones_hbm.at[pl.ds(0, K)], ones_v)
        plsc.subcore_barrier()

        def scatters(g, issue):
            for b in range(GROUP):
                d = pltpu.make_async_copy(
                    ones_v, acc_sh.at[didx.at[g * GROUP + b]], ssem)
                d.start(add=True) if issue else d.wait()

        def body(g, carry):
            scatters(g, True)
            scatters(g, False)
            return carry

        lax.fori_loop(0, NG, body, 0)
        plsc.subcore_barrier()
        _rowwise_copy(s, lambda o, n: acc_sh.at[pl.ds(o, n)],
                      lambda o, n: out_hbm.at[pl.ds(c * N + o, n)])

    return _sc_degree


RING = 5


@functools.cache
def _make_sc_prop(F, k):
    nchunk = EPW // k
    assert nchunk % RING == 0
    mesh = plsc.VectorSubcoreMesh(core_axis_name="c", subcore_axis_name="s")

    @functools.partial(
        pl.kernel,
        out_type=jax.ShapeDtypeStruct((2 * N, F), jnp.float32),
        mesh=mesh,
        scratch_types=[
            pltpu.VMEM((nchunk, k), jnp.int32),
            pltpu.VMEM((nchunk, k), jnp.int32),
            pltpu.VMEM((RING, k, F), jnp.float32),
            pltpu.VMEM_SHARED((N, F), jnp.float32),
            [pltpu.SemaphoreType.DMA] * RING,
            [pltpu.SemaphoreType.DMA] * RING,
        ],
        compiler_params=pltpu.CompilerParams(use_tc_tiling_on_sc=False),
    )
    def _sc_prop(feat_hbm, src3_hbm, dst3_hbm, out_hbm,
                 sidx, didx, rows, acc_sh, gsems, ssems):
        c = lax.axis_index("c")
        s = lax.axis_index("s")
        wid = c * NS + s
        pltpu.sync_copy(src3_hbm.at[wid], sidx)
        pltpu.sync_copy(dst3_hbm.at[wid], didx)
        _rowwise_copy(s, lambda o, n: feat_hbm.at[pl.ds(o, n)],
                      lambda o, n: acc_sh.at[pl.ds(o, n)])
        plsc.subcore_barrier()

        def gather(b, chunk):
            return pltpu.make_async_copy(
                feat_hbm.at[sidx.at[chunk]], rows.at[b], gsems[b])

        def scatter(b, chunk):
            return pltpu.make_async_copy(
                rows.at[b], acc_sh.at[didx.at[chunk]], ssems[b])

        for b in range(RING):
            gather(b, b).start()

        def body(i, carry):
            for b in range(RING):
                chunk = i * RING + b
                gather(b, chunk).wait()
                scatter(b, chunk).start(add=True)

                @pl.when(chunk + RING < nchunk)
                def _():
                    scatter(b, chunk).wait()
                    gather(b, chunk + RING).start()
            return carry

        lax.fori_loop(0, nchunk // RING, body, 0)
        for b in range(RING):
            scatter(b, nchunk - RING + b).wait()
        plsc.subcore_barrier()
        _rowwise_copy(s, lambda o, n: acc_sh.at[pl.ds(o, n)],
                      lambda o, n: out_hbm.at[pl.ds(c * N + o, n)])

    return _sc_prop


_B = 2000


def _tc_prep_body(deg_ref, x_ref, d_ref, x1_ref):
    deg = deg_ref[0] + deg_ref[1] - 1.0
    d = lax.rsqrt(deg)
    d_ref[...] = d
    x1_ref[...] = x_ref[...] * d[:, :1]


def _tc_prep(deg_par, x):
    return pl.pallas_call(
        _tc_prep_body,
        grid=(N // _B,),
        in_specs=[
            pl.BlockSpec((2, _B, 8), lambda i: (0, i, 0)),
            pl.BlockSpec((_B, F_IN), lambda i: (i, 0)),
        ],
        out_specs=[
            pl.BlockSpec((_B, 8), lambda i: (i, 0)),
            pl.BlockSpec((_B, F_IN), lambda i: (i, 0)),
        ],
        out_shape=[
            jax.ShapeDtypeStruct((N, 8), jnp.float32),
            jax.ShapeDtypeStruct((N, F_IN), jnp.float32),
        ],
    )(deg_par, x)


def _tc_mid_body(p1_ref, x1_ref, d_ref, W1_ref, b1_ref, W2_ref, t2_ref):
    d = d_ref[:, :1]
    p1 = (p1_ref[0] + p1_ref[1] - x1_ref[...]) * d
    h1 = jnp.dot(p1, W1_ref[...], preferred_element_type=jnp.float32)
    h1 = jnp.maximum(h1 + b1_ref[...], 0.0)
    t2 = jnp.dot(h1, W2_ref[...], preferred_element_type=jnp.float32)
    t2_ref[...] = t2 * d


def _tc_mid(p1_par, x1, d8, W1, b1, W2):
    return pl.pallas_call(
        _tc_mid_body,
        grid=(N // _B,),
        in_specs=[
            pl.BlockSpec((2, _B, F_IN), lambda i: (0, i, 0)),
            pl.BlockSpec((_B, F_IN), lambda i: (i, 0)),
            pl.BlockSpec((_B, 8), lambda i: (i, 0)),
            pl.BlockSpec((F_IN, F_HID), lambda i: (0, 0)),
            pl.BlockSpec((1, F_HID), lambda i: (0, 0)),
            pl.BlockSpec((F_HID, F_OUT), lambda i: (0, 0)),
        ],
        out_specs=pl.BlockSpec((_B, F_OUT), lambda i: (i, 0)),
        out_shape=jax.ShapeDtypeStruct((N, F_OUT), jnp.float32),
    )(p1_par, x1, d8, W1, b1.reshape(1, F_HID), W2)


def _tc_final_body(p2_ref, t2_ref, d_ref, b2_ref, out_ref):
    d = d_ref[:, :1]
    p2 = (p2_ref[0] + p2_ref[1] - t2_ref[...]) * d + b2_ref[...]
    m = jnp.max(p2, axis=1, keepdims=True)
    lse = jnp.log(jnp.sum(jnp.exp(p2 - m), axis=1, keepdims=True))
    out_ref[...] = p2 - m - lse


def _tc_final(p2_par, t2, d8, b2):
    return pl.pallas_call(
        _tc_final_body,
        grid=(N // _B,),
        in_specs=[
            pl.BlockSpec((2, _B, F_OUT), lambda i: (0, i, 0)),
            pl.BlockSpec((_B, F_OUT), lambda i: (i, 0)),
            pl.BlockSpec((_B, 8), lambda i: (i, 0)),
            pl.BlockSpec((1, F_OUT), lambda i: (0, 0)),
        ],
        out_specs=pl.BlockSpec((_B, F_OUT), lambda i: (i, 0)),
        out_shape=jax.ShapeDtypeStruct((N, F_OUT), jnp.float32),
    )(p2_par, t2, d8, b2.reshape(1, F_OUT))


def kernel(x, block, W1, b1, W2, b2):
    src40 = block[0].reshape(NW, EPW // 40, 40)
    dst40 = block[1].reshape(NW, EPW // 40, 40)
    src80 = block[0].reshape(NW, EPW // 80, 80)
    dst80 = block[1].reshape(NW, EPW // 80, 80)
    ones8 = jnp.ones((N, 8), dtype=jnp.float32)

    deg_par = _get_sc_degree()(dst40, ones8).reshape(2, N, 8)
    d8, x1 = _tc_prep(deg_par, x)
    p1_par = _make_sc_prop(F_IN, 40)(x1, src40, dst40).reshape(2, N, F_IN)
    t2 = _tc_mid(p1_par, x1, d8, W1, b1, W2)
    p2_par = _make_sc_prop(F_OUT, 80)(t2, src80, dst80).reshape(2, N, F_OUT)
    return _tc_final(p2_par, t2, d8, b2)


# speedup vs baseline: 46.7993x; 1.0485x over previous
import functools

import jax
import jax.numpy as jnp
from jax import lax
from jax.experimental import pallas as pl
from jax.experimental.pallas import tpu as pltpu
from jax.experimental.pallas import tpu_sc as plsc

N = 10000
E = 320000
F_IN = 128
F_HID = 256
F_OUT = 64

NC = 2
NS = 16
NW = NC * NS
EPW = E // NW
K = 40
NCHUNK = EPW // K
GROUP = 5
NG = NCHUNK // GROUP
R0 = 624
R1 = N - 15 * R0


def _rowwise_copy(s, mk_src, mk_dst):
    @pl.when(s < 15)
    def _():
        pltpu.sync_copy(mk_src(s * R0, R0), mk_dst(s * R0, R0))

    @pl.when(s == 15)
    def _():
        pltpu.sync_copy(mk_src(15 * R0, R1), mk_dst(15 * R0, R1))

@functools.cache
def _get_sc_degree():
    mesh = plsc.VectorSubcoreMesh(core_axis_name="c", subcore_axis_name="s")

    @functools.partial(
        pl.kernel,
        out_type=jax.ShapeDtypeStruct((2 * N, 8), jnp.float32),
        mesh=mesh,
        scratch_types=[
            pltpu.VMEM((NCHUNK, K), jnp.int32),
            pltpu.VMEM((K, 8), jnp.float32),
            pltpu.VMEM_SHARED((N, 8), jnp.float32),
            pltpu.SemaphoreType.DMA,
        ],
        compiler_params=pltpu.CompilerParams(use_tc_tiling_on_sc=False),
    )
    def _sc_degree(dst3_hbm, ones_hbm, out_hbm, didx, ones_v, acc_sh, ssem):
        c = lax.axis_index("c")
        s = lax.axis_index("s")
        wid = c * NS + s
        pltpu.sync_copy(dst3_hbm.at[wid], didx)
        _rowwise_copy(s, lambda o, n: ones_hbm.at[pl.ds(o, n)],
                      lambda o, n: acc_sh.at[pl.ds(o, n)])
        pltpu.sync_copy(ones_hbm.at[pl.ds(0, K)], ones_v)
        plsc.subcore_barrier()

        def scatters(g, issue):
            for b in range(GROUP):
                d = pltpu.make_async_copy(
                    ones_v, acc_sh.at[didx.at[g * GROUP + b]], ssem)
                d.start(add=True) if issue else d.wait()

        def body(g, carry):
            scatters(g, True)
            scatters(g, False)
            return carry

        lax.fori_loop(0, NG, body, 0)
        plsc.subcore_barrier()
        _rowwise_copy(s, lambda o, n: acc_sh.at[pl.ds(o, n)],
                      lambda o, n: out_hbm.at[pl.ds(c * N + o, n)])

    return _sc_degree


RING = 5


@functools.cache
def _make_sc_prop(F, k, dtype):
    nchunk = EPW // k
    assert nchunk % RING == 0
    mesh = plsc.VectorSubcoreMesh(core_axis_name="c", subcore_axis_name="s")

    @functools.partial(
        pl.kernel,
        out_type=jax.ShapeDtypeStruct((2 * N, F), dtype),
        mesh=mesh,
        scratch_types=[
            pltpu.VMEM((nchunk, k), jnp.int32),
            pltpu.VMEM((nchunk, k), jnp.int32),
            pltpu.VMEM((RING, k, F), dtype),
            pltpu.VMEM_SHARED((N, F), dtype),
            [pltpu.SemaphoreType.DMA] * RING,
            [pltpu.SemaphoreType.DMA] * RING,
        ],
        compiler_params=pltpu.CompilerParams(use_tc_tiling_on_sc=False),
    )
    def _sc_prop(feat_hbm, src3_hbm, dst3_hbm, out_hbm,
                 sidx, didx, rows, acc_sh, gsems, ssems):
        c = lax.axis_index("c")
        s = lax.axis_index("s")
        wid = c * NS + s
        pltpu.sync_copy(src3_hbm.at[wid], sidx)
        pltpu.sync_copy(dst3_hbm.at[wid], didx)
        _rowwise_copy(s, lambda o, n: feat_hbm.at[pl.ds(o, n)],
                      lambda o, n: acc_sh.at[pl.ds(o, n)])
        plsc.subcore_barrier()

        def gather(b, chunk):
            return pltpu.make_async_copy(
                feat_hbm.at[sidx.at[chunk]], rows.at[b], gsems[b])

        def scatter(b, chunk):
            return pltpu.make_async_copy(
                rows.at[b], acc_sh.at[didx.at[chunk]], ssems[b])

        for b in range(RING):
            gather(b, b).start()

        def body(i, carry):
            for b in range(RING):
                chunk = i * RING + b
                gather(b, chunk).wait()
                scatter(b, chunk).start(add=True)

                @pl.when(chunk + RING < nchunk)
                def _():
                    scatter(b, chunk).wait()
                    gather(b, chunk + RING).start()
            return carry

        lax.fori_loop(0, nchunk // RING, body, 0)
        for b in range(RING):
            scatter(b, nchunk - RING + b).wait()
        plsc.subcore_barrier()
        _rowwise_copy(s, lambda o, n: acc_sh.at[pl.ds(o, n)],
                      lambda o, n: out_hbm.at[pl.ds(c * N + o, n)])

    return _sc_prop


_B = 2000


def _tc_prep_body(deg_ref, x_ref, d_ref, x1_ref):
    deg = deg_ref[0] + deg_ref[1] - 1.0
    d = lax.rsqrt(deg)
    d_ref[...] = d
    x1_ref[...] = (x_ref[...] * d[:, :1]).astype(jnp.bfloat16)


def _tc_prep(deg_par, x):
    return pl.pallas_call(
        _tc_prep_body,
        grid=(N // _B,),
        in_specs=[
            pl.BlockSpec((2, _B, 8), lambda i: (0, i, 0)),
            pl.BlockSpec((_B, F_IN), lambda i: (i, 0)),
        ],
        out_specs=[
            pl.BlockSpec((_B, 8), lambda i: (i, 0)),
            pl.BlockSpec((_B, F_IN), lambda i: (i, 0)),
        ],
        out_shape=[
            jax.ShapeDtypeStruct((N, 8), jnp.float32),
            jax.ShapeDtypeStruct((N, F_IN), jnp.bfloat16),
        ],
    )(deg_par, x)


def _tc_mid_body(p1_ref, x1_ref, d_ref, W1_ref, b1_ref, W2_ref, t2_ref):
    d = d_ref[:, :1]
    p1 = (p1_ref[0].astype(jnp.float32) + p1_ref[1].astype(jnp.float32)
          - x1_ref[...].astype(jnp.float32)) * d
    h1 = jnp.dot(p1, W1_ref[...], preferred_element_type=jnp.float32)
    h1 = jnp.maximum(h1 + b1_ref[...], 0.0)
    t2 = jnp.dot(h1, W2_ref[...], preferred_element_type=jnp.float32)
    t2_ref[...] = t2 * d


def _tc_mid(p1_par, x1, d8, W1, b1, W2):
    return pl.pallas_call(
        _tc_mid_body,
        grid=(N // _B,),
        in_specs=[
            pl.BlockSpec((2, _B, F_IN), lambda i: (0, i, 0)),
            pl.BlockSpec((_B, F_IN), lambda i: (i, 0)),
            pl.BlockSpec((_B, 8), lambda i: (i, 0)),
            pl.BlockSpec((F_IN, F_HID), lambda i: (0, 0)),
            pl.BlockSpec((1, F_HID), lambda i: (0, 0)),
            pl.BlockSpec((F_HID, F_OUT), lambda i: (0, 0)),
        ],
        out_specs=pl.BlockSpec((_B, F_OUT), lambda i: (i, 0)),
        out_shape=jax.ShapeDtypeStruct((N, F_OUT), jnp.float32),
    )(p1_par, x1, d8, W1, b1.reshape(1, F_HID), W2)


def _tc_final_body(p2_ref, t2_ref, d_ref, b2_ref, out_ref):
    d = d_ref[:, :1]
    p2 = (p2_ref[0] + p2_ref[1] - t2_ref[...]) * d + b2_ref[...]
    m = jnp.max(p2, axis=1, keepdims=True)
    lse = jnp.log(jnp.sum(jnp.exp(p2 - m), axis=1, keepdims=True))
    out_ref[...] = p2 - m - lse


def _tc_final(p2_par, t2, d8, b2):
    return pl.pallas_call(
        _tc_final_body,
        grid=(N // _B,),
        in_specs=[
            pl.BlockSpec((2, _B, F_OUT), lambda i: (0, i, 0)),
            pl.BlockSpec((_B, F_OUT), lambda i: (i, 0)),
            pl.BlockSpec((_B, 8), lambda i: (i, 0)),
            pl.BlockSpec((1, F_OUT), lambda i: (0, 0)),
        ],
        out_specs=pl.BlockSpec((_B, F_OUT), lambda i: (i, 0)),
        out_shape=jax.ShapeDtypeStruct((N, F_OUT), jnp.float32),
    )(p2_par, t2, d8, b2.reshape(1, F_OUT))


def kernel(x, block, W1, b1, W2, b2):
    dst40 = block[1].reshape(NW, EPW // 40, 40)
    src80 = block[0].reshape(NW, EPW // 80, 80)
    dst80 = block[1].reshape(NW, EPW // 80, 80)
    ones8 = jnp.ones((N, 8), dtype=jnp.float32)

    deg_par = _get_sc_degree()(dst40, ones8).reshape(2, N, 8)
    d8, x1 = _tc_prep(deg_par, x)
    p1_par = _make_sc_prop(F_IN, 80, jnp.bfloat16)(
        x1, src80, dst80).reshape(2, N, F_IN)
    t2 = _tc_mid(p1_par, x1, d8, W1, b1, W2)
    p2_par = _make_sc_prop(F_OUT, 80, jnp.float32)(
        t2, src80, dst80).reshape(2, N, F_OUT)
    return _tc_final(p2_par, t2, d8, b2)


# speedup vs baseline: 49.7179x; 1.0624x over previous
import functools

import jax
import jax.numpy as jnp
from jax import lax
from jax.experimental import pallas as pl
from jax.experimental.pallas import tpu as pltpu
from jax.experimental.pallas import tpu_sc as plsc

N = 10000
E = 320000
F_IN = 128
F_HID = 256
F_OUT = 64

NC = 2
NS = 16
NW = NC * NS
EPW = E // NW
K = 40
NCHUNK = EPW // K
GROUP = 5
NG = NCHUNK // GROUP
R0 = 624
R1 = N - 15 * R0


def _rowwise_copy(s, mk_src, mk_dst):
    @pl.when(s < 15)
    def _():
        pltpu.sync_copy(mk_src(s * R0, R0), mk_dst(s * R0, R0))

    @pl.when(s == 15)
    def _():
        pltpu.sync_copy(mk_src(15 * R0, R1), mk_dst(15 * R0, R1))

@functools.cache
def _get_sc_degree():
    mesh = plsc.VectorSubcoreMesh(core_axis_name="c", subcore_axis_name="s")

    @functools.partial(
        pl.kernel,
        out_type=jax.ShapeDtypeStruct((2 * N, 8), jnp.float32),
        mesh=mesh,
        scratch_types=[
            pltpu.VMEM((NCHUNK, K), jnp.int32),
            pltpu.VMEM((K, 8), jnp.float32),
            pltpu.VMEM_SHARED((N, 8), jnp.float32),
            pltpu.SemaphoreType.DMA,
        ],
        compiler_params=pltpu.CompilerParams(use_tc_tiling_on_sc=False),
    )
    def _sc_degree(dst3_hbm, ones_hbm, out_hbm, didx, ones_v, acc_sh, ssem):
        c = lax.axis_index("c")
        s = lax.axis_index("s")
        wid = c * NS + s
        pltpu.sync_copy(dst3_hbm.at[wid], didx)
        _rowwise_copy(s, lambda o, n: ones_hbm.at[pl.ds(o, n)],
                      lambda o, n: acc_sh.at[pl.ds(o, n)])
        pltpu.sync_copy(ones_hbm.at[pl.ds(0, K)], ones_v)
        plsc.subcore_barrier()

        def scatters(g, issue):
            for b in range(GROUP):
                d = pltpu.make_async_copy(
                    ones_v, acc_sh.at[didx.at[g * GROUP + b]], ssem)
                d.start(add=True) if issue else d.wait()

        def body(g, carry):
            scatters(g, True)
            scatters(g, False)
            return carry

        lax.fori_loop(0, NG, body, 0)
        plsc.subcore_barrier()
        _rowwise_copy(s, lambda o, n: acc_sh.at[pl.ds(o, n)],
                      lambda o, n: out_hbm.at[pl.ds(c * N + o, n)])

    return _sc_degree


RING = 5


@functools.cache
def _make_sc_prop(F, k, dtype):
    nchunk = EPW // k
    assert nchunk % RING == 0
    mesh = plsc.VectorSubcoreMesh(core_axis_name="c", subcore_axis_name="s")

    @functools.partial(
        pl.kernel,
        out_type=jax.ShapeDtypeStruct((2 * N, F), dtype),
        mesh=mesh,
        scratch_types=[
            pltpu.VMEM((nchunk, k), jnp.int32),
            pltpu.VMEM((nchunk, k), jnp.int32),
            pltpu.VMEM((RING, k, F), dtype),
            pltpu.VMEM_SHARED((N, F), dtype),
            [pltpu.SemaphoreType.DMA] * RING,
            [pltpu.SemaphoreType.DMA] * RING,
        ],
        compiler_params=pltpu.CompilerParams(use_tc_tiling_on_sc=False),
    )
    def _sc_prop(feat_hbm, src3_hbm, dst3_hbm, out_hbm,
                 sidx, didx, rows, acc_sh, gsems, ssems):
        c = lax.axis_index("c")
        s = lax.axis_index("s")
        wid = c * NS + s
        pltpu.sync_copy(src3_hbm.at[wid], sidx)
        pltpu.sync_copy(dst3_hbm.at[wid], didx)
        _rowwise_copy(s, lambda o, n: feat_hbm.at[pl.ds(o, n)],
                      lambda o, n: acc_sh.at[pl.ds(o, n)])
        plsc.subcore_barrier()

        def gather(b, chunk):
            return pltpu.make_async_copy(
                feat_hbm.at[sidx.at[chunk]], rows.at[b], gsems[b])

        def scatter(b, chunk):
            return pltpu.make_async_copy(
                rows.at[b], acc_sh.at[didx.at[chunk]], ssems[b])

        for b in range(RING):
            gather(b, b).start()

        def body(i, carry):
            for b in range(RING):
                chunk = i * RING + b
                gather(b, chunk).wait()
                scatter(b, chunk).start(add=True)

                @pl.when(chunk + RING < nchunk)
                def _():
                    scatter(b, chunk).wait()
                    gather(b, chunk + RING).start()
            return carry

        lax.fori_loop(0, nchunk // RING, body, 0)
        for b in range(RING):
            scatter(b, nchunk - RING + b).wait()
        plsc.subcore_barrier()
        _rowwise_copy(s, lambda o, n: acc_sh.at[pl.ds(o, n)],
                      lambda o, n: out_hbm.at[pl.ds(c * N + o, n)])

    return _sc_prop


_B = 2000


def _tc_prep_body(deg_ref, x_ref, d_ref, x1_ref):
    deg = deg_ref[0] + deg_ref[1] - 1.0
    d = lax.rsqrt(deg)
    d_ref[...] = d
    x1_ref[...] = (x_ref[...] * d[:, :1]).astype(jnp.bfloat16)


def _tc_prep(deg_par, x):
    return pl.pallas_call(
        _tc_prep_body,
        grid=(N // _B,),
        in_specs=[
            pl.BlockSpec((2, _B, 8), lambda i: (0, i, 0)),
            pl.BlockSpec((_B, F_IN), lambda i: (i, 0)),
        ],
        out_specs=[
            pl.BlockSpec((_B, 8), lambda i: (i, 0)),
            pl.BlockSpec((_B, F_IN), lambda i: (i, 0)),
        ],
        out_shape=[
            jax.ShapeDtypeStruct((N, 8), jnp.float32),
            jax.ShapeDtypeStruct((N, F_IN), jnp.bfloat16),
        ],
    )(deg_par, x)


def _tc_mid_body(p1_ref, x1_ref, d_ref, W1_ref, b1_ref, W2_ref, t2_ref):
    d = d_ref[:, :1]
    p1 = (p1_ref[0].astype(jnp.float32) + p1_ref[1].astype(jnp.float32)
          - x1_ref[...].astype(jnp.float32)) * d
    h1 = jnp.dot(p1, W1_ref[...], preferred_element_type=jnp.float32)
    h1 = jnp.maximum(h1 + b1_ref[...], 0.0)
    t2 = jnp.dot(h1, W2_ref[...], preferred_element_type=jnp.float32)
    t2_ref[...] = (t2 * d).astype(jnp.bfloat16)


def _tc_mid(p1_par, x1, d8, W1, b1, W2):
    return pl.pallas_call(
        _tc_mid_body,
        grid=(N // _B,),
        in_specs=[
            pl.BlockSpec((2, _B, F_IN), lambda i: (0, i, 0)),
            pl.BlockSpec((_B, F_IN), lambda i: (i, 0)),
            pl.BlockSpec((_B, 8), lambda i: (i, 0)),
            pl.BlockSpec((F_IN, F_HID), lambda i: (0, 0)),
            pl.BlockSpec((1, F_HID), lambda i: (0, 0)),
            pl.BlockSpec((F_HID, F_OUT), lambda i: (0, 0)),
        ],
        out_specs=pl.BlockSpec((_B, F_OUT), lambda i: (i, 0)),
        out_shape=jax.ShapeDtypeStruct((N, F_OUT), jnp.bfloat16),
    )(p1_par, x1, d8, W1, b1.reshape(1, F_HID), W2)


def _tc_final_body(p2_ref, t2_ref, d_ref, b2_ref, out_ref):
    d = d_ref[:, :1]
    p2 = (p2_ref[0].astype(jnp.float32) + p2_ref[1].astype(jnp.float32)
          - t2_ref[...].astype(jnp.float32)) * d + b2_ref[...]
    m = jnp.max(p2, axis=1, keepdims=True)
    lse = jnp.log(jnp.sum(jnp.exp(p2 - m), axis=1, keepdims=True))
    out_ref[...] = p2 - m - lse


def _tc_final(p2_par, t2, d8, b2):
    return pl.pallas_call(
        _tc_final_body,
        grid=(N // _B,),
        in_specs=[
            pl.BlockSpec((2, _B, F_OUT), lambda i: (0, i, 0)),
            pl.BlockSpec((_B, F_OUT), lambda i: (i, 0)),
            pl.BlockSpec((_B, 8), lambda i: (i, 0)),
            pl.BlockSpec((1, F_OUT), lambda i: (0, 0)),
        ],
        out_specs=pl.BlockSpec((_B, F_OUT), lambda i: (i, 0)),
        out_shape=jax.ShapeDtypeStruct((N, F_OUT), jnp.float32),
    )(p2_par, t2, d8, b2.reshape(1, F_OUT))


def kernel(x, block, W1, b1, W2, b2):
    dst40 = block[1].reshape(NW, EPW // 40, 40)
    src80 = block[0].reshape(NW, EPW // 80, 80)
    dst80 = block[1].reshape(NW, EPW // 80, 80)
    ones8 = jnp.ones((N, 8), dtype=jnp.float32)

    deg_par = _get_sc_degree()(dst40, ones8).reshape(2, N, 8)
    d8, x1 = _tc_prep(deg_par, x)
    p1_par = _make_sc_prop(F_IN, 80, jnp.bfloat16)(
        x1, src80, dst80).reshape(2, N, F_IN)
    t2 = _tc_mid(p1_par, x1, d8, W1, b1, W2)
    p2_par = _make_sc_prop(F_OUT, 80, jnp.bfloat16)(
        t2, src80, dst80).reshape(2, N, F_OUT)
    return _tc_final(p2_par, t2, d8, b2)
